# baseline (speedup 1.0000x reference)
import functools

import numpy as np
import jax
import jax.numpy as jnp
from jax import lax
from jax.experimental import pallas as pl
from jax.experimental.pallas import tpu as pltpu
from jax.experimental.pallas import tpu_sc as plsc

NCH = 16
ACT_NORM = 1.6791767


def _mlp_body(xp_ref, k1_ref, k2_ref, k3_ref, k4_ref, out_ref):
    h = xp_ref[...]
    h = jax.nn.silu(jnp.dot(h, k1_ref[...], preferred_element_type=jnp.float32)) * ACT_NORM
    h = jax.nn.silu(jnp.dot(h, k2_ref[...], preferred_element_type=jnp.float32)) * ACT_NORM
    h = jax.nn.silu(jnp.dot(h, k3_ref[...], preferred_element_type=jnp.float32)) * ACT_NORM
    out_ref[...] = jnp.dot(h, k4_ref[...], preferred_element_type=jnp.float32)


def _run_mlp(x_packed, K1, K2, K3, K4, block):
    R = x_packed.shape[0]
    grid = (R // block,)
    return pl.pallas_call(
        _mlp_body,
        grid=grid,
        in_specs=[
            pl.BlockSpec((block, 128), lambda i: (i, 0)),
            pl.BlockSpec((128, 96), lambda i: (0, 0)),
            pl.BlockSpec((96, 96), lambda i: (0, 0)),
            pl.BlockSpec((96, 96), lambda i: (0, 0)),
            pl.BlockSpec((96, 768), lambda i: (0, 0)),
        ],
        out_specs=pl.BlockSpec((block, 768), lambda i: (i, 0)),
        out_shape=jax.ShapeDtypeStruct((R, 768), jnp.float32),
    )(x_packed, K1, K2, K3, K4)


def _lin_body(m_ref, wb_ref, o_ref):
    o_ref[...] = jnp.dot(m_ref[...], wb_ref[...], preferred_element_type=jnp.float32)


def _run_linear(msg, Wb, block):
    N = msg.shape[0]
    grid = (N // block,)
    return pl.pallas_call(
        _lin_body,
        grid=grid,
        in_specs=[
            pl.BlockSpec((block, 144), lambda i: (i, 0)),
            pl.BlockSpec((144, 144), lambda i: (0, 0)),
        ],
        out_specs=pl.BlockSpec((block, 144), lambda i: (i, 0)),
        out_shape=jax.ShapeDtypeStruct((N, 144), jnp.float32),
    )(msg, Wb)


_NTILES = 16
_NCHUNKS = 6
_W = 400
_C = 80


@functools.lru_cache(maxsize=None)
def _build_sc(E, NQ):
    EPT = E // _NTILES
    NW = EPT // _W
    WR = _W // 16
    ROWS_OUT = NQ // _NTILES
    CB = _W + _C + 16

    mesh = plsc.VectorSubcoreMesh(core_axis_name="c", subcore_axis_name="s")

    @functools.partial(
        pl.kernel,
        out_type=jax.ShapeDtypeStruct((_NCHUNKS * NQ, 144), jnp.float32),
        mesh=mesh,
        scratch_types=[
            pltpu.VMEM((_W,), jnp.int32),
            pltpu.VMEM((_W,), jnp.int32),
            pltpu.VMEM((_W * 9 + 16,), jnp.float32),
            pltpu.VMEM((WR, 768), jnp.float32),
            pltpu.VMEM((CB + 16,), jnp.int32),
            pltpu.VMEM((CB + 16,), jnp.int32),
            pltpu.VMEM((CB + 16,), jnp.int32),
            pltpu.VMEM((_C, 16), jnp.float32),
            pltpu.VMEM((_C // 16, 16), jnp.int32),
            pltpu.VMEM((_C, 144), jnp.float32),
            pltpu.VMEM_SHARED((NQ + 16, 144), jnp.float32),
            pltpu.SemaphoreType.DMA,
            pltpu.SemaphoreType.DMA,
        ],
        compiler_params=pltpu.CompilerParams(use_tc_tiling_on_sc=False,
                                             needs_layout_passes=False),
    )
    def sc_kernel(wp_hbm, ea_hbm, snd_hbm, rcv_hbm, nf_hbm, out_hbm,
                  rcv_v, snd_v, ea_v, w_v, lids_v, locs_v, snds_v,
                  xs_c, lidx_v, mij_v, acc, semL, semX):
        c = lax.axis_index("c")
        s = lax.axis_index("s")
        zeros16 = jnp.zeros((16,), jnp.float32)
        zeros16i = jnp.zeros((16,), jnp.int32)
        iota16 = lax.iota(jnp.int32, 16)

        def run_pass(p):
            q = 2 * p + c
            base_node = q * NQ

            def zrow(e, carry):
                for k in range(9):
                    mij_v[e, pl.ds(k * 16, 16)] = zeros16
                return carry
            lax.fori_loop(0, _C, zrow, 0)
            done = 0
            while done < ROWS_OUT:
                n = min(_C, ROWS_OUT - done)
                pltpu.sync_copy(mij_v.at[pl.ds(0, n)],
                                acc.at[pl.ds(s * ROWS_OUT + done, n)])
                done += n
            plsc.subcore_barrier()

            def window(wd, carry):
                e0 = s * EPT + wd * _W
                d1 = pltpu.async_copy(rcv_hbm.at[pl.ds(e0, _W)], rcv_v, semL)
                d2 = pltpu.async_copy(snd_hbm.at[pl.ds(e0, _W)], snd_v, semL)
                d3 = pltpu.async_copy(ea_hbm.at[pl.ds(e0 * 9, _W * 9)],
                                      ea_v.at[pl.ds(0, _W * 9)], semL)
                d4 = pltpu.async_copy(wp_hbm.at[pl.ds(e0 // 16, WR)], w_v, semL)
                d1.wait(); d2.wait(); d3.wait(); d4.wait()

                def scan_grp(g, cnt):
                    r = rcv_v[pl.ds(g * 16, 16)]
                    sn = snd_v[pl.ds(g * 16, 16)]
                    loc = r - base_node
                    m = (loc >= 0) & (loc < NQ)
                    lid = iota16 + g * 16
                    mi = jnp.where(m, jnp.int32(1), jnp.int32(0))
                    incl = plsc.cumsum(mi)
                    dest = jnp.where(m, cnt + incl - mi, CB + iota16)
                    plsc.store_scatter(lids_v, [dest], lid)
                    plsc.store_scatter(locs_v, [dest], loc)
                    plsc.store_scatter(snds_v, [dest], sn)
                    return cnt + incl[15]
                cnt = lax.fori_loop(0, _W // 16, scan_grp, jnp.int32(0))

                for k in range(_C // 16):
                    lids_v[pl.ds(cnt + k * 16, 16)] = zeros16i
                    locs_v[pl.ds(cnt + k * 16, 16)] = jnp.full((16,), NQ, jnp.int32)
                    snds_v[pl.ds(cnt + k * 16, 16)] = zeros16i
                nchunks = (cnt + (_C - 1)) // _C

                def chunk(j, carry2):
                    off = j * _C
                    pltpu.async_copy(nf_hbm.at[snds_v.at[pl.ds(off, _C)]],
                                     xs_c, semX).wait()
                    for g in range(_C // 16):
                        lid16 = lids_v[pl.ds(off + g * 16, 16)]
                        lidx_v[g, :] = locs_v[pl.ds(off + g * 16, 16)]
                        for j2 in range(16):
                            le = lid16[j2]
                            e = g * 16 + j2
                            xsr = xs_c[e, :]
                            row = le // 16
                            cb = (le % 16) * 48
                            xw0 = xsr * w_v[row, pl.ds(cb, 16)]
                            xw1 = xsr * w_v[row, pl.ds(cb + 16, 16)]
                            xw2 = xsr * w_v[row, pl.ds(cb + 32, 16)]
                            sh = ea_v[pl.ds(le * 9, 16)]
                            xws = (xw0, xw1, xw1, xw1, xw2, xw2, xw2, xw2, xw2)
                            for k in range(9):
                                mij_v[e, pl.ds(k * 16, 16)] = xws[k] * sh[k]
                    for g in range(_C // 16):
                        pltpu.sync_copy(mij_v.at[pl.ds(g * 16, 16)],
                                        acc.at[lidx_v.at[g]], add=True)
                    return carry2
                lax.fori_loop(0, nchunks, chunk, 0)
                return carry
            lax.fori_loop(0, NW, window, 0)
            plsc.subcore_barrier()

            pltpu.sync_copy(acc.at[pl.ds(s * ROWS_OUT, ROWS_OUT)],
                            out_hbm.at[pl.ds(q * NQ + s * ROWS_OUT, ROWS_OUT)])
            plsc.subcore_barrier()

        run_pass(0)
        run_pass(1)
        run_pass(2)

    return sc_kernel


def kernel(lenght, node_features, edge_attributes, edge_index,
           W_fc1, W_fc2, W_fc3, W_fc4, W_l0, W_l1, W_l2):
    E = lenght.shape[0]
    N = node_features.shape[0]

    NQ = ((N + _NCHUNKS * 128 - 1) // (_NCHUNKS * 128)) * 128

    eye = jnp.eye(16, dtype=jnp.float32)
    K1 = jnp.kron(eye, W_fc1 / np.sqrt(W_fc1.shape[0]))
    K2 = jnp.kron(eye, W_fc2 / np.sqrt(W_fc2.shape[0]))
    K3 = jnp.kron(eye, W_fc3 / np.sqrt(W_fc3.shape[0]))
    K4 = jnp.kron(eye, W_fc4 / np.sqrt(W_fc4.shape[0]))
    x_packed = lenght.reshape(E // 16, 128)
    wp = _run_mlp(x_packed, K1, K2, K3, K4, block=1000)

    snd = edge_index[0]
    rcv = edge_index[1]
    ea_flat = edge_attributes.reshape(E * 9)
    msg_pad = _build_sc(E, NQ)(wp, ea_flat, snd, rcv, node_features)
    msg = msg_pad[:N]

    inv = 1.0 / np.sqrt(NCH)
    Wb = jnp.zeros((144, 144), jnp.float32)
    Wb = Wb.at[0:16, 0:16].set(W_l0 * inv)
    for ci in range(3):
        Wb = Wb.at[16 * (1 + ci):16 * (2 + ci), 16 + ci:64:3].set(W_l1 * inv)
    for ci in range(5):
        Wb = Wb.at[16 * (4 + ci):16 * (5 + ci), 64 + ci:144:5].set(W_l2 * inv)
    return _run_linear(msg, Wb, block=1000)
